# Initial kernel scaffold; baseline (speedup 1.0000x reference)
import jax
import jax.numpy as jnp
from jax.experimental import pallas as pl


def kernel(x, edge_index, edge_attr, params):
    raise NotImplementedError("write your pallas kernel here")


# speedup vs baseline: 1.5435x; 1.5435x over previous
import functools

import jax
import jax.numpy as jnp
from jax import lax
from jax.experimental import pallas as pl
from jax.experimental.pallas import tpu as pltpu
from jax.experimental.pallas import tpu_sc as plsc

N = 10000
E = 160000
D_EDGE = 16
H = 32
L_LAYERS = 4
MLP_H = 128

NC = 2
NS = 16
NW = NC * NS

CH = 128
CHUNKS = 40
EPT = CH * CHUNKS
PAD_E = NW * EPT
NROWS = 10112
RPT = NROWS // NS

BE = 1024
NBLK = PAD_E // BE

_MESH = plsc.VectorSubcoreMesh(core_axis_name="c", subcore_axis_name="s")
_SC_PARAMS = pltpu.CompilerParams(use_tc_tiling_on_sc=False)


@functools.partial(
    pl.kernel,
    out_type=(
        jax.ShapeDtypeStruct((NC, NROWS, H), jnp.float32),
        jax.ShapeDtypeStruct((NC, NROWS, H), jnp.float32),
    ),
    mesh=_MESH,
    scratch_types=[
        pltpu.VMEM((CH,), jnp.int32),
        pltpu.VMEM((CH,), jnp.int32),
        pltpu.VMEM((CH, H), jnp.float32),
        pltpu.VMEM((CH, H), jnp.float32),
        pltpu.VMEM((CH, H), jnp.float32),
        pltpu.VMEM((RPT, H), jnp.float32),
        pltpu.VMEM_SHARED((NROWS, H), jnp.float32),
        pltpu.VMEM_SHARED((NROWS, H), jnp.float32),
        pltpu.SemaphoreType.DMA,
    ],
    compiler_params=_SC_PARAMS,
)
def _sc_messages(e_hbm, v_hbm, row_hbm, col_hbm, seg1_hbm, seg2_hbm,
                 rowb, colb, eb, vb, m2b, zrows, acc1, acc2, sem):
    c = lax.axis_index("c")
    s = lax.axis_index("s")
    wid = c * NS + s

    def zrow(j, _):
        zero = jnp.zeros((16,), jnp.float32)
        zrows[j, pl.ds(0, 16)] = zero
        zrows[j, pl.ds(16, 16)] = zero
        return 0
    lax.fori_loop(0, RPT, zrow, 0)
    pltpu.sync_copy(zrows, acc1.at[pl.ds(s * RPT, RPT)])
    pltpu.sync_copy(zrows, acc2.at[pl.ds(s * RPT, RPT)])
    plsc.subcore_barrier()

    base = wid * EPT

    def chunk(t, _):
        off = pl.multiple_of(base + t * CH, CH)
        pltpu.sync_copy(row_hbm.at[pl.ds(off, CH)], rowb)
        pltpu.sync_copy(col_hbm.at[pl.ds(off, CH)], colb)
        pltpu.sync_copy(e_hbm.at[pl.ds(off, CH)], eb)
        pltpu.async_copy(v_hbm.at[rowb], vb, sem).wait()

        def rowop(j, _):
            for half in (0, 16):
                ev = eb[j, pl.ds(half, 16)]
                vv = vb[j, pl.ds(half, 16)]
                vb[j, pl.ds(half, 16)] = jnp.maximum(vv + ev, 0.0) + 1e-7
                m2b[j, pl.ds(half, 16)] = jnp.maximum(ev, 0.0) + 1e-7
            return 0
        lax.fori_loop(0, CH, rowop, 0)

        pltpu.sync_copy(vb, acc1.at[colb], add=True)
        pltpu.sync_copy(m2b, acc2.at[colb], add=True)
        return 0
    lax.fori_loop(0, CHUNKS, chunk, 0)
    plsc.subcore_barrier()

    pltpu.sync_copy(acc1.at[pl.ds(s * RPT, RPT)], seg1_hbm.at[c, pl.ds(s * RPT, RPT)])
    pltpu.sync_copy(acc2.at[pl.ds(s * RPT, RPT)], seg2_hbm.at[c, pl.ds(s * RPT, RPT)])


@functools.partial(
    pl.kernel,
    out_type=jax.ShapeDtypeStruct((PAD_E, H), jnp.float32),
    mesh=_MESH,
    scratch_types=[
        pltpu.VMEM((CH,), jnp.int32),
        pltpu.VMEM((CH, H), jnp.float32),
        pltpu.SemaphoreType.DMA,
    ],
    compiler_params=_SC_PARAMS,
)
def _sc_gather(table_hbm, idx_hbm, out_hbm, idxb, rowsb, sem):
    c = lax.axis_index("c")
    s = lax.axis_index("s")
    base = (c * NS + s) * EPT

    def chunk(t, _):
        off = pl.multiple_of(base + t * CH, CH)
        pltpu.sync_copy(idx_hbm.at[pl.ds(off, CH)], idxb)
        pltpu.async_copy(table_hbm.at[idxb], rowsb, sem).wait()
        pltpu.sync_copy(rowsb, out_hbm.at[pl.ds(off, CH)])
        return 0
    lax.fori_loop(0, CHUNKS, chunk, 0)


def _dense_mlp_kernel(x_ref, w1_ref, b1_ref, g_ref, beta_ref, w2_ref, b2_ref, o_ref):
    h = jnp.dot(x_ref[...], w1_ref[...], preferred_element_type=jnp.float32, precision=lax.Precision.HIGHEST) + b1_ref[...]
    mu = jnp.mean(h, axis=0, keepdims=True)
    var = jnp.mean((h - mu) * (h - mu), axis=0, keepdims=True)
    hn = (h - mu) * lax.rsqrt(var + 1e-5) * g_ref[...] + beta_ref[...]
    o_ref[...] = jnp.dot(jnp.maximum(hn, 0.0), w2_ref[...],
                         preferred_element_type=jnp.float32, precision=lax.Precision.HIGHEST) + b2_ref[...]


def _mlp_small(xa, p):
    dout = p["W2"].shape[1]
    return pl.pallas_call(
        _dense_mlp_kernel,
        out_shape=jax.ShapeDtypeStruct((xa.shape[0], dout), jnp.float32),
    )(xa, p["W1"], p["b1"].reshape(1, -1), p["g"].reshape(1, -1),
      p["beta"].reshape(1, -1), p["W2"], p["b2"].reshape(1, -1))


def _stats_kernel(nrows, x_ref, w1_ref, b1_ref, s_ref, q_ref, c_ref):
    i = pl.program_id(0)
    h = jnp.dot(x_ref[...], w1_ref[...], preferred_element_type=jnp.float32, precision=lax.Precision.HIGHEST) + b1_ref[...]

    @pl.when(i == 0)
    def _():
        c_ref[...] = jnp.mean(h, axis=0, keepdims=True)
        s_ref[...] = jnp.zeros_like(s_ref)
        q_ref[...] = jnp.zeros_like(q_ref)

    rows = i * BE + lax.broadcasted_iota(jnp.int32, (BE, 1), 0)
    d = jnp.where(rows < nrows, h - c_ref[...], 0.0)
    s_ref[...] += jnp.sum(d, axis=0, keepdims=True)
    q_ref[...] += jnp.sum(d * d, axis=0, keepdims=True)


def _stats_call(xa, w1, b1, nrows):
    din = xa.shape[1]
    return pl.pallas_call(
        functools.partial(_stats_kernel, nrows),
        grid=(NBLK,),
        in_specs=[
            pl.BlockSpec((BE, din), lambda i: (i, 0)),
            pl.BlockSpec((din, MLP_H), lambda i: (0, 0)),
            pl.BlockSpec((1, MLP_H), lambda i: (0, 0)),
        ],
        out_specs=[
            pl.BlockSpec((1, MLP_H), lambda i: (0, 0)),
            pl.BlockSpec((1, MLP_H), lambda i: (0, 0)),
            pl.BlockSpec((1, MLP_H), lambda i: (0, 0)),
        ],
        out_shape=[
            jax.ShapeDtypeStruct((1, MLP_H), jnp.float32),
            jax.ShapeDtypeStruct((1, MLP_H), jnp.float32),
            jax.ShapeDtypeStruct((1, MLP_H), jnp.float32),
        ],
    )(xa, w1, b1.reshape(1, -1))


def _apply_kernel(nrows, x_ref, w1_ref, b1_ref, g_ref, beta_ref, w2_ref, b2_ref,
                  s_ref, q_ref, c_ref, o_ref):
    h = jnp.dot(x_ref[...], w1_ref[...], preferred_element_type=jnp.float32, precision=lax.Precision.HIGHEST) + b1_ref[...]
    ds = s_ref[...] * (1.0 / nrows)
    mu = c_ref[...] + ds
    var = q_ref[...] * (1.0 / nrows) - ds * ds
    hn = (h - mu) * lax.rsqrt(var + 1e-5) * g_ref[...] + beta_ref[...]
    o_ref[...] = jnp.dot(jnp.maximum(hn, 0.0), w2_ref[...],
                         preferred_element_type=jnp.float32, precision=lax.Precision.HIGHEST) + b2_ref[...]


def _apply_call(xa, p, stats, nrows):
    din = xa.shape[1]
    dout = p["W2"].shape[1]
    return pl.pallas_call(
        functools.partial(_apply_kernel, nrows),
        grid=(NBLK,),
        in_specs=[
            pl.BlockSpec((BE, din), lambda i: (i, 0)),
            pl.BlockSpec((din, MLP_H), lambda i: (0, 0)),
            pl.BlockSpec((1, MLP_H), lambda i: (0, 0)),
            pl.BlockSpec((1, MLP_H), lambda i: (0, 0)),
            pl.BlockSpec((1, MLP_H), lambda i: (0, 0)),
            pl.BlockSpec((MLP_H, dout), lambda i: (0, 0)),
            pl.BlockSpec((1, dout), lambda i: (0, 0)),
            pl.BlockSpec((1, MLP_H), lambda i: (0, 0)),
            pl.BlockSpec((1, MLP_H), lambda i: (0, 0)),
            pl.BlockSpec((1, MLP_H), lambda i: (0, 0)),
        ],
        out_specs=pl.BlockSpec((BE, dout), lambda i: (i, 0)),
        out_shape=jax.ShapeDtypeStruct((PAD_E, dout), jnp.float32),
    )(xa, p["W1"], p["b1"].reshape(1, -1), p["g"].reshape(1, -1),
      p["beta"].reshape(1, -1), p["W2"], p["b2"].reshape(1, -1),
      stats[0], stats[1], stats[2])


BN = 1264


def _node_update_kernel(v_ref, s1_ref, s2_ref, wn_ref, bn_ref, we_ref,
                        vo_ref, z_ref):
    seg1 = s1_ref[0] + s1_ref[1]
    vo_ref[...] = v_ref[...] + jnp.maximum(
        jnp.dot(v_ref[...] + seg1, wn_ref[...],
                preferred_element_type=jnp.float32, precision=lax.Precision.HIGHEST) + bn_ref[...], 0.0)
    seg2 = s2_ref[0] + s2_ref[1]
    z_ref[...] = jnp.dot(seg2, we_ref[...], preferred_element_type=jnp.float32, precision=lax.Precision.HIGHEST)


def _node_update(v, seg1, seg2, wn, bn, we):
    return pl.pallas_call(
        _node_update_kernel,
        grid=(NROWS // BN,),
        in_specs=[
            pl.BlockSpec((BN, H), lambda i: (i, 0)),
            pl.BlockSpec((2, BN, H), lambda i: (0, i, 0)),
            pl.BlockSpec((2, BN, H), lambda i: (0, i, 0)),
            pl.BlockSpec((H, H), lambda i: (0, 0)),
            pl.BlockSpec((1, H), lambda i: (0, 0)),
            pl.BlockSpec((H, H), lambda i: (0, 0)),
        ],
        out_specs=(
            pl.BlockSpec((BN, H), lambda i: (i, 0)),
            pl.BlockSpec((BN, H), lambda i: (i, 0)),
        ),
        out_shape=(
            jax.ShapeDtypeStruct((NROWS, H), jnp.float32),
            jax.ShapeDtypeStruct((NROWS, H), jnp.float32),
        ),
    )(v, seg1, seg2, wn, bn.reshape(1, -1), we)


def _edge_update_kernel(e_ref, gz_ref, we_ref, be_ref, o_ref):
    o_ref[...] = e_ref[...] + jnp.maximum(
        jnp.dot(e_ref[...], we_ref[...], preferred_element_type=jnp.float32, precision=lax.Precision.HIGHEST)
        + gz_ref[...] + be_ref[...], 0.0)


def _edge_update(e, gz, we, be):
    return pl.pallas_call(
        _edge_update_kernel,
        grid=(NBLK,),
        in_specs=[
            pl.BlockSpec((BE, H), lambda i: (i, 0)),
            pl.BlockSpec((BE, H), lambda i: (i, 0)),
            pl.BlockSpec((H, H), lambda i: (0, 0)),
            pl.BlockSpec((1, H), lambda i: (0, 0)),
        ],
        out_specs=pl.BlockSpec((BE, H), lambda i: (i, 0)),
        out_shape=jax.ShapeDtypeStruct((PAD_E, H), jnp.float32),
    )(e, gz, we, be.reshape(1, -1))


def kernel(x, edge_index, edge_attr, params):
    p = params
    ei = edge_index.astype(jnp.int32)
    row = ei[0]
    col = ei[1]
    perm = jnp.argsort(row * N + col, stable=True).astype(jnp.int32)

    pad = PAD_E - E
    row_p = jnp.concatenate([row, jnp.zeros((pad,), jnp.int32)])
    col_p = jnp.concatenate([col, jnp.full((pad,), N, jnp.int32)])
    perm_p = jnp.concatenate([perm, jnp.zeros((pad,), jnp.int32)])
    ea_p = jnp.concatenate(
        [edge_attr.astype(jnp.float32), jnp.zeros((pad, D_EDGE), jnp.float32)])

    v = jnp.concatenate(
        [_mlp_small(x, p["enc_n"]), jnp.zeros((NROWS - N, H), jnp.float32)])
    e = _apply_call(ea_p, p["enc_e"],
                    _stats_call(ea_p, p["enc_e"]["W1"], p["enc_e"]["b1"], E),
                    E)

    for l in range(L_LAYERS):
        seg1, seg2 = _sc_messages(e, v, row_p, col_p)
        v, z = _node_update(v, seg1, seg2, p["Wn"][l], p["bn"][l], p["We"][l])
        gz = _sc_gather(z, row_p)
        e = _edge_update(e, gz, p["We"][l], p["be"][l])

    node_out = _mlp_small(v[:N], p["dec_n"])

    es = _sc_gather(e, perm_p)
    edge_out = _apply_call(es, p["dec_e"],
                           _stats_call(e, p["dec_e"]["W1"], p["dec_e"]["b1"], E),
                           E)
    return node_out, edge_out[:E]
